# baseline (device time: 198997 ns/iter reference)
import numpy as np
import jax
import jax.numpy as jnp
from jax import lax
from jax.experimental import pallas as pl
from jax.experimental.pallas import tpu as pltpu

N_DEV = 32
FWD_HOPS = N_DEV // 2
BWD_HOPS = N_DEV // 2 - 1
NSLOT = 3

_PLANE = [(0, 0), (1, 0), (1, 1), (0, 1), (0, 2), (1, 2), (1, 3), (0, 3)]
_LOGICAL_COORDS = [(x, y, z) for z in range(4) for (x, y) in _PLANE]
_C2L = {c: i for i, c in enumerate(_LOGICAL_COORDS)}

_RING = []
for z in range(4):
    ys = range(4) if z % 2 == 0 else range(3, -1, -1)
    _RING.extend((0, y, z) for y in ys)
for z in range(3, -1, -1):
    ys = range(4) if z % 2 == 1 else range(3, -1, -1)
    _RING.extend((1, y, z) for y in ys)
assert len(set(_RING)) == N_DEV
for _i in range(N_DEV):
    _a, _b = _RING[_i], _RING[(_i + 1) % N_DEV]
    assert sum(abs(p - q) for p, q in zip(_a, _b)) == 1, (_a, _b)

_RING_L = [_C2L[c] for c in _RING]
_POS = [0] * N_DEV
for _p, _l in enumerate(_RING_L):
    _POS[_l] = _p

def _flip(c, axis, bit):
    c = list(c)
    c[axis] ^= bit
    return tuple(c)

_FLIPS = [(0, 1), (1, 1), (1, 2), (2, 1), (2, 2)]

_TAB = np.zeros((N_DEV, 39), dtype=np.int32)
for _l in range(N_DEV):
    _p = _POS[_l]
    _TAB[_l, 0] = _RING_L[(_p + 1) % N_DEV]
    _TAB[_l, 1] = _RING_L[(_p - 1) % N_DEV]
    for _r, (_ax, _bit) in enumerate(_FLIPS):
        _TAB[_l, 2 + _r] = _C2L[_flip(_LOGICAL_COORDS[_l], _ax, _bit)]
    for _h in range(FWD_HOPS + 1):
        _TAB[_l, 7 + _h] = _RING_L[(_p - _h) % N_DEV]
    for _h in range(1, BWD_HOPS + 1):
        _TAB[_l, 24 + _h - 1] = _RING_L[(_p + _h) % N_DEV]
_TAB_J = jnp.asarray(_TAB)


def kernel(x, w_mat):
    m_per, k = x.shape
    _, n_per = w_mat.shape
    m_sub = m_per // 2

    def body(idx_ref, x_ref, w_ref, out_ref, wb_ref,
             comm_f, send_sems_f, recv_sems_f, credit_f0, credit_f1,
             comm_b, send_sems_b, recv_sems_b, credit_b0, credit_b1,
             maxsend_ref, maxcomm_ref, max_send_sems, max_recv_sem):
        credit_f = [credit_f0, credit_f1]
        credit_b = [credit_b0, credit_b1]
        my = lax.axis_index("i")
        succ = idx_ref[0]
        pred = idx_ref[1]

        barrier_sem = pltpu.get_barrier_semaphore()
        for nbr in [pred, succ]:
            pl.semaphore_signal(
                barrier_sem, inc=1,
                device_id=(nbr,), device_id_type=pl.DeviceIdType.MESH,
            )
        pl.semaphore_wait(barrier_sem, 2)

        xb = x_ref[:, :].astype(jnp.bfloat16)
        for u in range(2):
            sub = xb[u * m_sub:(u + 1) * m_sub, :]
            comm_f[0 * 2 + u] = sub
            comm_b[0 * 2 + u] = sub

        def mk(comm, send_sems, recv_sems, h, u, target):
            return pltpu.make_async_remote_copy(
                src_ref=comm.at[(h % NSLOT) * 2 + u],
                dst_ref=comm.at[((h + 1) % NSLOT) * 2 + u],
                send_sem=send_sems.at[u * NSLOT + h % NSLOT],
                recv_sem=recv_sems.at[u * NSLOT + (h + 1) % NSLOT],
                device_id=(target,), device_id_type=pl.DeviceIdType.MESH,
            )

        def gemm_sub(chunk, origin, u):
            yb = jnp.maximum(
                jnp.dot(chunk, wb_ref[:, :],
                        preferred_element_type=jnp.float32),
                0.0,
            )
            out_ref[pl.ds(origin * m_per + u * m_sub, m_sub), :] = yb
            return jnp.max(yb)

        prev_f = [mk(comm_f, send_sems_f, recv_sems_f, 0, u, succ)
                  for u in range(2)]
        prev_b = [mk(comm_b, send_sems_b, recv_sems_b, 0, u, pred)
                  for u in range(2)]
        for u in range(2):
            prev_f[u].start()
            prev_b[u].start()

        wb_ref[:, :] = w_ref[:, :].astype(jnp.bfloat16)
        maxcomm_ref[:, :, :] = jnp.zeros(maxcomm_ref.shape, jnp.float32)

        m = jnp.float32(0.0)
        for u in range(2):
            m = jnp.maximum(m, gemm_sub(comm_f[0 * 2 + u], idx_ref[7], u))

        for h in range(1, FWD_HOPS - 1):
            for u in range(2):
                if h >= 2:
                    pl.semaphore_wait(credit_f[u], 1)
                prev_f[u].wait_send()
                pl.semaphore_signal(
                    credit_f[u], inc=1,
                    device_id=(pred,), device_id_type=pl.DeviceIdType.MESH,
                )
                prev_f[u].wait_recv()
                cur = mk(comm_f, send_sems_f, recv_sems_f, h, u, succ)
                cur.start()
                prev_f[u] = cur
                if h >= 2:
                    pl.semaphore_wait(credit_b[u], 1)
                prev_b[u].wait_send()
                if h <= BWD_HOPS - 1:
                    pl.semaphore_signal(
                        credit_b[u], inc=1,
                        device_id=(succ,), device_id_type=pl.DeviceIdType.MESH,
                    )
                prev_b[u].wait_recv()
                curb = mk(comm_b, send_sems_b, recv_sems_b, h, u, pred)
                curb.start()
                prev_b[u] = curb
                m = jnp.maximum(
                    m, gemm_sub(comm_f[(h % NSLOT) * 2 + u],
                                idx_ref[7 + h], u))
                m = jnp.maximum(
                    m, gemm_sub(comm_b[(h % NSLOT) * 2 + u],
                                idx_ref[24 + h - 1], u))

        h = FWD_HOPS - 1
        pl.semaphore_wait(credit_f[0], 1)
        prev_f[0].wait_send()
        prev_f[0].wait_recv()
        cur = mk(comm_f, send_sems_f, recv_sems_f, h, 0, succ)
        cur.start()
        prev_f[0] = cur
        m = jnp.maximum(m, gemm_sub(comm_f[(h % NSLOT) * 2 + 0],
                                    idx_ref[7 + h], 0))
        pl.semaphore_wait(credit_f[1], 1)
        prev_f[1].wait_send()
        prev_f[1].wait_recv()
        m = jnp.maximum(m, gemm_sub(comm_f[(h % NSLOT) * 2 + 1],
                                    idx_ref[7 + h], 1))
        pl.semaphore_wait(credit_b[0], 1)
        prev_b[0].wait_send()
        prev_b[0].wait_recv()
        m = jnp.maximum(m, gemm_sub(comm_b[(h % NSLOT) * 2 + 0],
                                    idx_ref[24 + h - 1], 0))
        pl.semaphore_wait(credit_b[1], 1)
        prev_b[1].wait_send()
        prev_b[1].wait_recv()
        curb = mk(comm_b, send_sems_b, recv_sems_b, h, 1, pred)
        curb.start()
        prev_b[1] = curb
        m = jnp.maximum(m, gemm_sub(comm_b[(h % NSLOT) * 2 + 1],
                                    idx_ref[24 + h - 1], 1))

        fin_slot = (FWD_HOPS % NSLOT) * 2
        prev_f[0].wait_send()
        prev_f[0].wait_recv()
        m = jnp.maximum(m, gemm_sub(comm_f[fin_slot + 0],
                                    idx_ref[7 + FWD_HOPS], 0))
        prev_b[1].wait_send()
        prev_b[1].wait_recv()
        m = jnp.maximum(m, gemm_sub(comm_b[fin_slot + 1],
                                    idx_ref[7 + FWD_HOPS], 1))

        maxsend_ref[:, :] = jnp.broadcast_to(m, maxsend_ref.shape)
        allmax = []
        for d in range(1, N_DEV):
            target = lax.rem(my + d, N_DEV)
            rdma = pltpu.make_async_remote_copy(
                src_ref=maxsend_ref, dst_ref=maxcomm_ref.at[my],
                send_sem=max_send_sems.at[d - 1], recv_sem=max_recv_sem,
                device_id=(target,), device_id_type=pl.DeviceIdType.MESH,
            )
            rdma.start()
            allmax.append(rdma)
        for rdma in allmax:
            rdma.wait_recv()
        m = jnp.maximum(m, jnp.max(maxcomm_ref[:, :, :]))
        for rdma in allmax:
            rdma.wait_send()

        scale = m / 127.0
        y = out_ref[:, :]
        q = jnp.clip(jnp.round(y / scale), -127.0, 127.0)
        out_ref[:, :] = q * scale

    grid_spec = pltpu.PrefetchScalarGridSpec(
        num_scalar_prefetch=1,
        grid=(),
        in_specs=[
            pl.BlockSpec(memory_space=pltpu.VMEM),
            pl.BlockSpec(memory_space=pltpu.VMEM),
        ],
        out_specs=pl.BlockSpec(memory_space=pltpu.VMEM),
        scratch_shapes=[
            pltpu.VMEM((k, n_per), jnp.bfloat16),
            pltpu.VMEM((NSLOT * 2, m_per // 2, k), jnp.bfloat16),
            pltpu.SemaphoreType.DMA((2 * NSLOT,)),
            pltpu.SemaphoreType.DMA((2 * NSLOT,)),
            pltpu.SemaphoreType.REGULAR,
            pltpu.SemaphoreType.REGULAR,
            pltpu.VMEM((NSLOT * 2, m_per // 2, k), jnp.bfloat16),
            pltpu.SemaphoreType.DMA((2 * NSLOT,)),
            pltpu.SemaphoreType.DMA((2 * NSLOT,)),
            pltpu.SemaphoreType.REGULAR,
            pltpu.SemaphoreType.REGULAR,
            pltpu.VMEM((8, 128), jnp.float32),
            pltpu.VMEM((N_DEV, 8, 128), jnp.float32),
            pltpu.SemaphoreType.DMA((N_DEV - 1,)),
            pltpu.SemaphoreType.DMA,
        ],
    )
    idx = _TAB_J[lax.axis_index("i")]
    return pl.pallas_call(
        body,
        grid_spec=grid_spec,
        out_shape=jax.ShapeDtypeStruct((N_DEV * m_per, n_per), jnp.float32),
        compiler_params=pltpu.CompilerParams(collective_id=0),
    )(idx, x, w_mat)
